# baseline (speedup 1.0000x reference)
import functools

import jax
import jax.numpy as jnp
from jax import lax
from jax.experimental import pallas as pl
from jax.experimental.pallas import tpu as pltpu
from jax.experimental.pallas import tpu_sc as plsc

_N = 10000
_E = 320000
_B = 16
_K = 80
_CHUNKS = 504
_BLK = 12
_NBLK = _CHUNKS // _BLK
_SPAN = _CHUNKS * _K


def _mm0_body(x_ref, w_ref, b_ref, out_ref):
    out_ref[0] = lax.dot_general(
        x_ref[...], w_ref[0], (((1,), (1,)), ((), ())),
        preferred_element_type=jnp.float32) + b_ref[0]


def _mm0(x, w, b):
    return pl.pallas_call(
        _mm0_body,
        grid=(4,),
        in_specs=[
            pl.BlockSpec((_N, 128), lambda j: (0, 0)),
            pl.BlockSpec((1, 128, 128), lambda j: (j, 0, 0)),
            pl.BlockSpec((1, 1, 128), lambda j: (j, 0, 0)),
        ],
        out_specs=pl.BlockSpec((1, _N, 128), lambda j: (j, 0, 0)),
        out_shape=jax.ShapeDtypeStruct((4, _N, 128), jnp.float32),
    )(x, w, b)


def _mm1_body(pre_ref, w_ref, b_ref, out_ref):
    h0 = jnp.maximum(pre_ref[0], 0.0)
    h1 = jnp.maximum(pre_ref[1], 0.0)
    out_ref[0] = (
        lax.dot_general(h0, w_ref[0, 0], (((1,), (1,)), ((), ())),
                        preferred_element_type=jnp.float32)
        + lax.dot_general(h1, w_ref[0, 1], (((1,), (1,)), ((), ())),
                          preferred_element_type=jnp.float32)
        + b_ref[0])


def _mm1(pre, w, b):
    return pl.pallas_call(
        _mm1_body,
        grid=(4,),
        in_specs=[
            pl.BlockSpec((2, _N, 128), lambda j: (0, 0, 0)),
            pl.BlockSpec((1, 2, 128, 128), lambda j: (j, 0, 0, 0)),
            pl.BlockSpec((1, 1, 128), lambda j: (j, 0, 0)),
        ],
        out_specs=pl.BlockSpec((1, _N, 128), lambda j: (j, 0, 0)),
        out_shape=jax.ShapeDtypeStruct((4, _N, 128), jnp.float32),
    )(pre, w, b)


def _sc_scatter_body(table, glist, slist, out, gbuf0, gbuf1, sbuf0, sbuf1,
                     rows0, rows1, rows2, rows3, acc,
                     semi, semg0, semg1, semg2, semg3,
                     sems0, sems1, sems2, sems3):
    c = lax.axis_index("c")
    s = lax.axis_index("s")
    base_g = s * _SPAN
    voff = (c + 2) * _N
    r0 = s * 624

    @pl.when(s < 15)
    def _():
        pltpu.sync_copy(table.at[pl.ds(c * _N + r0, 624)],
                        acc.at[pl.ds(r0, 624)])

    @pl.when(s == 15)
    def _():
        pltpu.sync_copy(table.at[pl.ds(c * _N + 9360, 640)],
                        acc.at[pl.ds(9360, 640)])

    plsc.subcore_barrier()

    rows = (rows0, rows1, rows2, rows3)
    semg = (semg0, semg1, semg2, semg3)
    sems = (sems0, sems1, sems2, sems3)

    def _load_block(bidx, gb, sb):
        pltpu.async_copy(glist.at[pl.ds(base_g + bidx * (_BLK * _K),
                                        _BLK * _K)], gb, semi)
        pltpu.async_copy(slist.at[s, bidx], sb, semi)

    def _wait_block(bidx, gb, sb):
        pltpu.make_async_copy(glist.at[pl.ds(base_g + bidx * (_BLK * _K),
                                             _BLK * _K)], gb, semi).wait()
        pltpu.make_async_copy(slist.at[s, bidx], sb, semi).wait()
        for l in range(_BLK * _K // 16):
            gb[pl.ds(16 * l, 16)] = gb[pl.ds(16 * l, 16)] + voff

    def _block(u, b, gb, sb, gbn, sbn):
        for k in range(_BLK):
            q = k % 4
            qn = (q + 1) % 4
            if k >= 3:
                pltpu.make_async_copy(rows[qn], acc.at[sb.at[k - 3]],
                                      sems[qn]).wait()
            if k < _BLK - 1:
                pltpu.async_copy(table.at[gb.at[pl.ds((k + 1) * _K, _K)]],
                                 rows[qn], semg[qn])
            else:
                @pl.when((2 * u + b) < _NBLK - 1)
                def _():
                    _wait_block(2 * u + b + 1, gbn, sbn)
                    pltpu.async_copy(table.at[gbn.at[pl.ds(0, _K)]],
                                     rows[qn], semg[qn])
            pltpu.make_async_copy(table.at[gb.at[pl.ds(k * _K, _K)]],
                                  rows[q], semg[q]).wait()
            pltpu.async_copy(rows[q], acc.at[sb.at[k]], sems[q], add=True)
        for k in (_BLK - 3, _BLK - 2, _BLK - 1):
            q = k % 4
            pltpu.make_async_copy(rows[q], acc.at[sb.at[k]], sems[q]).wait()

    nblk2 = _NBLK // 2

    _load_block(0, gbuf0, sbuf0)
    _wait_block(0, gbuf0, sbuf0)
    pltpu.async_copy(table.at[gbuf0.at[pl.ds(0, _K)]], rows0, semg0)

    def step(u, carry):
        _load_block(2 * u + 1, gbuf1, sbuf1)
        _block(u, 0, gbuf0, sbuf0, gbuf1, sbuf1)

        @pl.when(u < nblk2 - 1)
        def _():
            _load_block(2 * u + 2, gbuf0, sbuf0)

        _block(u, 1, gbuf1, sbuf1, gbuf0, sbuf0)
        return carry

    lax.fori_loop(0, nblk2, step, 0)
    plsc.subcore_barrier()

    @pl.when(s < 15)
    def _():
        pltpu.sync_copy(acc.at[pl.ds(r0, 624)],
                        out.at[pl.ds(c * _N + r0, 624)])

    @pl.when(s == 15)
    def _():
        pltpu.sync_copy(acc.at[pl.ds(9360, 640)],
                        out.at[pl.ds(c * _N + 9360, 640)])


@functools.cache
def _get_sc_scatter():
    return pl.kernel(
        _sc_scatter_body,
        jax.ShapeDtypeStruct((2 * _N, 128), jnp.float32),
        mesh=plsc.VectorSubcoreMesh(core_axis_name="c", subcore_axis_name="s"),
        scratch_types=(
            [pltpu.VMEM((_BLK * _K,), jnp.int32)] * 2
            + [pltpu.VMEM((_BLK, _K), jnp.int32)] * 2
            + [pltpu.VMEM((_K, 128), jnp.float32)] * 4
            + [pltpu.VMEM_SHARED((_N + 16, 128), jnp.float32)]
            + [pltpu.SemaphoreType.DMA] * 9
        ),
    )


def _head_body(pre_ref, vidx_ref, w1_ref, b1_ref, w2_ref, b2_ref, out_ref,
               seg_ref, cnt_ref):
    i = pl.program_id(0)

    @pl.when(i == 0)
    def _():
        seg_ref[...] = jnp.zeros_like(seg_ref)
        cnt_ref[...] = jnp.zeros_like(cnt_ref)

    ids = vidx_ref[0]
    iot = lax.broadcasted_iota(jnp.int32, (_B, 1250), 0)
    maskf = (ids == iot).astype(jnp.float32)
    h0 = jnp.maximum(pre_ref[0, 0], 0.0)
    h1 = jnp.maximum(pre_ref[1, 0], 0.0)
    seg_ref[:, :128] += jnp.dot(maskf, h0, preferred_element_type=jnp.float32)
    seg_ref[:, 128:] += jnp.dot(maskf, h1, preferred_element_type=jnp.float32)
    cnt_ref[...] += jnp.broadcast_to(
        jnp.sum(maskf, axis=1, keepdims=True), (_B, 128))

    @pl.when(i == 7)
    def _():
        mean = seg_ref[...] / cnt_ref[:, :1]
        y = lax.dot_general(mean, w1_ref[...], (((1,), (1,)), ((), ())),
                            preferred_element_type=jnp.float32) + b1_ref[...]
        y = jnp.maximum(y, 0.0)
        z = lax.dot_general(y, w2_ref[...], (((1,), (1,)), ((), ())),
                            preferred_element_type=jnp.float32) + b2_ref[...]
        out_ref[...] = 1.0 / (1.0 + jnp.exp(-z))


def _head(pre, vidx, w1, b1, w2, b2):
    return pl.pallas_call(
        _head_body,
        grid=(8,),
        in_specs=[
            pl.BlockSpec((2, 1, 1250, 128), lambda i: (0, i, 0, 0)),
            pl.BlockSpec((1, 1, 1250), lambda i: (i, 0, 0)),
            pl.BlockSpec((1024, 256), lambda i: (0, 0)),
            pl.BlockSpec((1, 1024), lambda i: (0, 0)),
            pl.BlockSpec((128, 1024), lambda i: (0, 0)),
            pl.BlockSpec((1, 128), lambda i: (0, 0)),
        ],
        out_specs=pl.BlockSpec((_B, 128), lambda i: (0, 0)),
        out_shape=jax.ShapeDtypeStruct((_B, 128), jnp.float32),
        scratch_shapes=[
            pltpu.VMEM((_B, 256), jnp.float32),
            pltpu.VMEM((_B, 128), jnp.float32),
        ],
    )(pre, vidx, w1, b1, w2, b2)


def kernel(verts, edges, verts_idx, W0_0, b0_0, W1_0, b1_0, W0_1, b0_1,
           W1_1, b1_1, fc1_w, fc1_b, fc2_w, fc2_b):
    src = edges[:, 0].astype(jnp.int32)
    dst = edges[:, 1].astype(jnp.int32)
    pad = _SPAN - 2 * _E // 16
    g2 = jnp.concatenate([dst, src]).reshape(16, -1)
    glist = jnp.pad(g2, ((0, 0), (0, pad))).reshape(-1)
    s2 = jnp.concatenate([src, dst]).reshape(16, -1)
    dump = jnp.broadcast_to(
        _N + jnp.arange(16, dtype=jnp.int32)[:, None], (16, pad))
    slist = jnp.concatenate([s2, dump], axis=1).reshape(16, _NBLK, _BLK, _K)

    w_a = jnp.stack([W0_0[:128], W0_0[128:], W1_0[:128], W1_0[128:]])
    b_a = jnp.stack([b0_0[:128], b0_0[128:], b1_0[:128], b1_0[128:]])
    b_a = b_a.reshape(4, 1, 128)
    table0 = _mm0(verts, w_a, b_a)
    pre0 = _get_sc_scatter()(table0.reshape(4 * _N, 128), glist, slist)

    w_c = jnp.stack([
        jnp.stack([W0_1[:128, :128], W0_1[:128, 128:]]),
        jnp.stack([W0_1[128:, :128], W0_1[128:, 128:]]),
        jnp.stack([W1_1[:128, :128], W1_1[:128, 128:]]),
        jnp.stack([W1_1[128:, :128], W1_1[128:, 128:]]),
    ])
    b_c = jnp.stack([b0_1[:128], b0_1[128:], b1_1[:128], b1_1[128:]])
    b_c = b_c.reshape(4, 1, 128)
    table1 = _mm1(pre0.reshape(2, _N, 128), w_c, b_c)
    pre1 = _get_sc_scatter()(table1.reshape(4 * _N, 128), glist, slist)

    fc2_wp = jnp.pad(fc2_w, ((0, 118), (0, 0)))
    fc2_bp = jnp.pad(fc2_b, (0, 118)).reshape(1, 128)
    out = _head(pre1.reshape(2, 8, 1250, 128),
                verts_idx.reshape(8, 1, 1250).astype(jnp.int32),
                fc1_w, fc1_b.reshape(1, 1024), fc2_wp, fc2_bp)
    return out[:, :10]


# speedup vs baseline: 1.3981x; 1.3981x over previous
import functools

import jax
import jax.numpy as jnp
from jax import lax
from jax.experimental import pallas as pl
from jax.experimental.pallas import tpu as pltpu
from jax.experimental.pallas import tpu_sc as plsc

_N = 10000
_E = 320000
_B = 16
_K = 80
_CHUNKS = (2 * _E) // (16 * _K)
_BLK = 50


def _mm0_body(x_ref, w_ref, b_ref, out_ref):
    out_ref[0] = lax.dot_general(
        x_ref[...], w_ref[0], (((1,), (1,)), ((), ())),
        preferred_element_type=jnp.float32) + b_ref[0]


def _mm0(x, w, b):
    return pl.pallas_call(
        _mm0_body,
        grid=(4,),
        in_specs=[
            pl.BlockSpec((_N, 128), lambda j: (0, 0)),
            pl.BlockSpec((1, 128, 128), lambda j: (j, 0, 0)),
            pl.BlockSpec((1, 1, 128), lambda j: (j, 0, 0)),
        ],
        out_specs=pl.BlockSpec((1, _N, 128), lambda j: (j, 0, 0)),
        out_shape=jax.ShapeDtypeStruct((4, _N, 128), jnp.float32),
    )(x, w, b)


def _mm1_body(pre_ref, w_ref, b_ref, out_ref):
    h0 = jnp.maximum(pre_ref[0], 0.0)
    h1 = jnp.maximum(pre_ref[1], 0.0)
    out_ref[0] = (
        lax.dot_general(h0, w_ref[0, 0], (((1,), (1,)), ((), ())),
                        preferred_element_type=jnp.float32)
        + lax.dot_general(h1, w_ref[0, 1], (((1,), (1,)), ((), ())),
                          preferred_element_type=jnp.float32)
        + b_ref[0])


def _mm1(pre, w, b):
    return pl.pallas_call(
        _mm1_body,
        grid=(4,),
        in_specs=[
            pl.BlockSpec((2, _N, 128), lambda j: (0, 0, 0)),
            pl.BlockSpec((1, 2, 128, 128), lambda j: (j, 0, 0, 0)),
            pl.BlockSpec((1, 1, 128), lambda j: (j, 0, 0)),
        ],
        out_specs=pl.BlockSpec((1, _N, 128), lambda j: (j, 0, 0)),
        out_shape=jax.ShapeDtypeStruct((4, _N, 128), jnp.float32),
    )(pre, w, b)


def _sc_scatter_body(table, glist, slist, out, gbuf0, gbuf1, sbuf0, sbuf1,
                     rows0, rows1, acc, semi, semg0, semg1, sems0, sems1):
    c = lax.axis_index("c")
    s = lax.axis_index("s")
    base_g = s * (_CHUNKS * _K)
    voff = (c + 2) * _N
    r0 = s * 624

    @pl.when(s < 15)
    def _():
        pltpu.sync_copy(table.at[pl.ds(c * _N + r0, 624)],
                        acc.at[pl.ds(r0, 624)])

    @pl.when(s == 15)
    def _():
        pltpu.sync_copy(table.at[pl.ds(c * _N + 9360, 640)],
                        acc.at[pl.ds(9360, 640)])

    plsc.subcore_barrier()

    rows = (rows0, rows1)
    semg = (semg0, semg1)
    sems = (sems0, sems1)

    def _load_block(bidx, gb, sb):
        pltpu.async_copy(glist.at[pl.ds(base_g + bidx * (_BLK * _K),
                                        _BLK * _K)], gb, semi)
        pltpu.async_copy(slist.at[s, bidx], sb, semi)

    def _wait_block(bidx, gb, sb):
        pltpu.make_async_copy(glist.at[pl.ds(base_g + bidx * (_BLK * _K),
                                             _BLK * _K)], gb, semi).wait()
        pltpu.make_async_copy(slist.at[s, bidx], sb, semi).wait()
        for l in range(_BLK * _K // 16):
            gb[pl.ds(16 * l, 16)] = gb[pl.ds(16 * l, 16)] + voff

    def _block(u, b, gb, sb, gbn, sbn):
        for k in range(_BLK):
            r = (k + b * (_BLK % 2)) % 2
            rn = 1 - r
            if k > 0:
                pltpu.make_async_copy(rows[rn], acc.at[sb.at[k - 1]],
                                      sems[rn]).wait()
            if k < _BLK - 1:
                pltpu.async_copy(table.at[gb.at[pl.ds((k + 1) * _K, _K)]],
                                 rows[rn], semg[rn])
            else:
                @pl.when((2 * u + b) < (_CHUNKS // _BLK) - 1)
                def _():
                    _wait_block(2 * u + b + 1, gbn, sbn)
                    pltpu.async_copy(table.at[gbn.at[pl.ds(0, _K)]],
                                     rows[rn], semg[rn])
            pltpu.make_async_copy(table.at[gb.at[pl.ds(k * _K, _K)]],
                                  rows[r], semg[r]).wait()
            pltpu.async_copy(rows[r], acc.at[sb.at[k]], sems[r], add=True)
        rl = (_BLK - 1 + b * (_BLK % 2)) % 2
        pltpu.make_async_copy(rows[rl], acc.at[sb.at[_BLK - 1]],
                              sems[rl]).wait()

    nblk2 = _CHUNKS // (2 * _BLK)

    _load_block(0, gbuf0, sbuf0)
    _wait_block(0, gbuf0, sbuf0)
    pltpu.async_copy(table.at[gbuf0.at[pl.ds(0, _K)]], rows0, semg0)

    def step(u, carry):
        _load_block(2 * u + 1, gbuf1, sbuf1)
        _block(u, 0, gbuf0, sbuf0, gbuf1, sbuf1)

        @pl.when(u < nblk2 - 1)
        def _():
            _load_block(2 * u + 2, gbuf0, sbuf0)

        _block(u, 1, gbuf1, sbuf1, gbuf0, sbuf0)
        return carry

    lax.fori_loop(0, nblk2, step, 0)
    plsc.subcore_barrier()

    @pl.when(s < 15)
    def _():
        pltpu.sync_copy(acc.at[pl.ds(r0, 624)],
                        out.at[pl.ds(c * _N + r0, 624)])

    @pl.when(s == 15)
    def _():
        pltpu.sync_copy(acc.at[pl.ds(9360, 640)],
                        out.at[pl.ds(c * _N + 9360, 640)])


@functools.cache
def _get_sc_scatter():
    return pl.kernel(
        _sc_scatter_body,
        jax.ShapeDtypeStruct((2 * _N, 128), jnp.float32),
        mesh=plsc.VectorSubcoreMesh(core_axis_name="c", subcore_axis_name="s"),
        scratch_types=(
            [pltpu.VMEM((_BLK * _K,), jnp.int32)] * 2
            + [pltpu.VMEM((_BLK, _K), jnp.int32)] * 2
            + [pltpu.VMEM((_K, 128), jnp.float32)] * 2
            + [pltpu.VMEM_SHARED((_N, 128), jnp.float32)]
            + [pltpu.SemaphoreType.DMA] * 5
        ),
    )


def _head_body(pre_ref, vidx_ref, w1_ref, b1_ref, w2_ref, b2_ref, out_ref,
               seg_ref, cnt_ref):
    i = pl.program_id(0)

    @pl.when(i == 0)
    def _():
        seg_ref[...] = jnp.zeros_like(seg_ref)
        cnt_ref[...] = jnp.zeros_like(cnt_ref)

    ids = vidx_ref[0]
    iot = lax.broadcasted_iota(jnp.int32, (_B, 1250), 0)
    maskf = (ids == iot).astype(jnp.float32)
    h0 = jnp.maximum(pre_ref[0, 0], 0.0)
    h1 = jnp.maximum(pre_ref[1, 0], 0.0)
    seg_ref[:, :128] += jnp.dot(maskf, h0, preferred_element_type=jnp.float32)
    seg_ref[:, 128:] += jnp.dot(maskf, h1, preferred_element_type=jnp.float32)
    cnt_ref[...] += jnp.broadcast_to(
        jnp.sum(maskf, axis=1, keepdims=True), (_B, 128))

    @pl.when(i == 7)
    def _():
        mean = seg_ref[...] / cnt_ref[:, :1]
        y = lax.dot_general(mean, w1_ref[...], (((1,), (1,)), ((), ())),
                            preferred_element_type=jnp.float32) + b1_ref[...]
        y = jnp.maximum(y, 0.0)
        z = lax.dot_general(y, w2_ref[...], (((1,), (1,)), ((), ())),
                            preferred_element_type=jnp.float32) + b2_ref[...]
        out_ref[...] = 1.0 / (1.0 + jnp.exp(-z))


def _head(pre, vidx, w1, b1, w2, b2):
    return pl.pallas_call(
        _head_body,
        grid=(8,),
        in_specs=[
            pl.BlockSpec((2, 1, 1250, 128), lambda i: (0, i, 0, 0)),
            pl.BlockSpec((1, 1, 1250), lambda i: (i, 0, 0)),
            pl.BlockSpec((1024, 256), lambda i: (0, 0)),
            pl.BlockSpec((1, 1024), lambda i: (0, 0)),
            pl.BlockSpec((128, 1024), lambda i: (0, 0)),
            pl.BlockSpec((1, 128), lambda i: (0, 0)),
        ],
        out_specs=pl.BlockSpec((_B, 128), lambda i: (0, 0)),
        out_shape=jax.ShapeDtypeStruct((_B, 128), jnp.float32),
        scratch_shapes=[
            pltpu.VMEM((_B, 256), jnp.float32),
            pltpu.VMEM((_B, 128), jnp.float32),
        ],
    )(pre, vidx, w1, b1, w2, b2)


def kernel(verts, edges, verts_idx, W0_0, b0_0, W1_0, b1_0, W0_1, b0_1,
           W1_1, b1_1, fc1_w, fc1_b, fc2_w, fc2_b):
    src = edges[:, 0].astype(jnp.int32)
    dst = edges[:, 1].astype(jnp.int32)
    glist = jnp.concatenate([dst, src])
    slist = jnp.concatenate([src, dst]).reshape(16, _CHUNKS // _BLK,
                                                _BLK, _K)

    w_a = jnp.stack([W0_0[:128], W0_0[128:], W1_0[:128], W1_0[128:]])
    b_a = jnp.stack([b0_0[:128], b0_0[128:], b1_0[:128], b1_0[128:]])
    b_a = b_a.reshape(4, 1, 128)
    table0 = _mm0(verts, w_a, b_a)
    pre0 = _get_sc_scatter()(table0.reshape(4 * _N, 128), glist, slist)

    w_c = jnp.stack([
        jnp.stack([W0_1[:128, :128], W0_1[:128, 128:]]),
        jnp.stack([W0_1[128:, :128], W0_1[128:, 128:]]),
        jnp.stack([W1_1[:128, :128], W1_1[:128, 128:]]),
        jnp.stack([W1_1[128:, :128], W1_1[128:, 128:]]),
    ])
    b_c = jnp.stack([b0_1[:128], b0_1[128:], b1_1[:128], b1_1[128:]])
    b_c = b_c.reshape(4, 1, 128)
    table1 = _mm1(pre0.reshape(2, _N, 128), w_c, b_c)
    pre1 = _get_sc_scatter()(table1.reshape(4 * _N, 128), glist, slist)

    fc2_wp = jnp.pad(fc2_w, ((0, 118), (0, 0)))
    fc2_bp = jnp.pad(fc2_b, (0, 118)).reshape(1, 128)
    out = _head(pre1.reshape(2, 8, 1250, 128),
                verts_idx.reshape(8, 1, 1250).astype(jnp.int32),
                fc1_w, fc1_b.reshape(1, 1024), fc2_wp, fc2_bp)
    return out[:, :10]


# speedup vs baseline: 1.4040x; 1.0042x over previous
import functools

import jax
import jax.numpy as jnp
from jax import lax
from jax.experimental import pallas as pl
from jax.experimental.pallas import tpu as pltpu
from jax.experimental.pallas import tpu_sc as plsc

_N = 10000
_E = 320000
_B = 16
_K = 80
_CHUNKS = (2 * _E) // (16 * _K)
_BLK = 25


def _mm0_body(x_ref, w_ref, b_ref, out_ref):
    out_ref[0] = lax.dot_general(
        x_ref[...], w_ref[0], (((1,), (1,)), ((), ())),
        preferred_element_type=jnp.float32) + b_ref[0]


def _mm0(x, w, b):
    return pl.pallas_call(
        _mm0_body,
        grid=(4,),
        in_specs=[
            pl.BlockSpec((_N, 128), lambda j: (0, 0)),
            pl.BlockSpec((1, 128, 128), lambda j: (j, 0, 0)),
            pl.BlockSpec((1, 1, 128), lambda j: (j, 0, 0)),
        ],
        out_specs=pl.BlockSpec((1, _N, 128), lambda j: (j, 0, 0)),
        out_shape=jax.ShapeDtypeStruct((4, _N, 128), jnp.float32),
    )(x, w, b)


def _mm1_body(pre_ref, w_ref, b_ref, out_ref):
    h0 = jnp.maximum(pre_ref[0], 0.0)
    h1 = jnp.maximum(pre_ref[1], 0.0)
    out_ref[0] = (
        lax.dot_general(h0, w_ref[0, 0], (((1,), (1,)), ((), ())),
                        preferred_element_type=jnp.float32)
        + lax.dot_general(h1, w_ref[0, 1], (((1,), (1,)), ((), ())),
                          preferred_element_type=jnp.float32)
        + b_ref[0])


def _mm1(pre, w, b):
    return pl.pallas_call(
        _mm1_body,
        grid=(4,),
        in_specs=[
            pl.BlockSpec((2, _N, 128), lambda j: (0, 0, 0)),
            pl.BlockSpec((1, 2, 128, 128), lambda j: (j, 0, 0, 0)),
            pl.BlockSpec((1, 1, 128), lambda j: (j, 0, 0)),
        ],
        out_specs=pl.BlockSpec((1, _N, 128), lambda j: (j, 0, 0)),
        out_shape=jax.ShapeDtypeStruct((4, _N, 128), jnp.float32),
    )(pre, w, b)


def _sc_scatter_body(table, glist, slist, out, gbuf0, gbuf1, sbuf0, sbuf1,
                     rows0, rows1, acc, semi, semg0, semg1, sems0, sems1):
    c = lax.axis_index("c")
    s = lax.axis_index("s")
    base_g = s * (_CHUNKS * _K)
    voff = (c + 2) * _N
    r0 = s * 624

    @pl.when(s < 15)
    def _():
        pltpu.sync_copy(table.at[pl.ds(c * _N + r0, 624)],
                        acc.at[pl.ds(r0, 624)])

    @pl.when(s == 15)
    def _():
        pltpu.sync_copy(table.at[pl.ds(c * _N + 9360, 640)],
                        acc.at[pl.ds(9360, 640)])

    plsc.subcore_barrier()

    rows = (rows0, rows1)
    semg = (semg0, semg1)
    sems = (sems0, sems1)

    def _load_block(bidx, gb, sb):
        pltpu.async_copy(glist.at[pl.ds(base_g + bidx * (_BLK * _K),
                                        _BLK * _K)], gb, semi)
        pltpu.async_copy(slist.at[s, bidx], sb, semi)

    def _wait_block(bidx, gb, sb):
        pltpu.make_async_copy(glist.at[pl.ds(base_g + bidx * (_BLK * _K),
                                             _BLK * _K)], gb, semi).wait()
        pltpu.make_async_copy(slist.at[s, bidx], sb, semi).wait()
        for l in range(_BLK * _K // 16):
            gb[pl.ds(16 * l, 16)] = gb[pl.ds(16 * l, 16)] + voff

    def _block(u, b, gb, sb, gbn, sbn):
        for k in range(_BLK):
            r = (k + b * (_BLK % 2)) % 2
            rn = 1 - r
            if k > 0:
                pltpu.make_async_copy(rows[rn], acc.at[sb.at[k - 1]],
                                      sems[rn]).wait()
            if k < _BLK - 1:
                pltpu.async_copy(table.at[gb.at[pl.ds((k + 1) * _K, _K)]],
                                 rows[rn], semg[rn])
            else:
                @pl.when((2 * u + b) < (_CHUNKS // _BLK) - 1)
                def _():
                    _wait_block(2 * u + b + 1, gbn, sbn)
                    pltpu.async_copy(table.at[gbn.at[pl.ds(0, _K)]],
                                     rows[rn], semg[rn])
            pltpu.make_async_copy(table.at[gb.at[pl.ds(k * _K, _K)]],
                                  rows[r], semg[r]).wait()
            pltpu.async_copy(rows[r], acc.at[sb.at[k]], sems[r], add=True)
        rl = (_BLK - 1 + b * (_BLK % 2)) % 2
        pltpu.make_async_copy(rows[rl], acc.at[sb.at[_BLK - 1]],
                              sems[rl]).wait()

    nblk2 = _CHUNKS // (2 * _BLK)

    _load_block(0, gbuf0, sbuf0)
    _wait_block(0, gbuf0, sbuf0)
    pltpu.async_copy(table.at[gbuf0.at[pl.ds(0, _K)]], rows0, semg0)

    def step(u, carry):
        _load_block(2 * u + 1, gbuf1, sbuf1)
        _block(u, 0, gbuf0, sbuf0, gbuf1, sbuf1)

        @pl.when(u < nblk2 - 1)
        def _():
            _load_block(2 * u + 2, gbuf0, sbuf0)

        _block(u, 1, gbuf1, sbuf1, gbuf0, sbuf0)
        return carry

    lax.fori_loop(0, nblk2, step, 0)
    plsc.subcore_barrier()

    @pl.when(s < 15)
    def _():
        pltpu.sync_copy(acc.at[pl.ds(r0, 624)],
                        out.at[pl.ds(c * _N + r0, 624)])

    @pl.when(s == 15)
    def _():
        pltpu.sync_copy(acc.at[pl.ds(9360, 640)],
                        out.at[pl.ds(c * _N + 9360, 640)])


@functools.cache
def _get_sc_scatter():
    return pl.kernel(
        _sc_scatter_body,
        jax.ShapeDtypeStruct((2 * _N, 128), jnp.float32),
        mesh=plsc.VectorSubcoreMesh(core_axis_name="c", subcore_axis_name="s"),
        scratch_types=(
            [pltpu.VMEM((_BLK * _K,), jnp.int32)] * 2
            + [pltpu.VMEM((_BLK, _K), jnp.int32)] * 2
            + [pltpu.VMEM((_K, 128), jnp.float32)] * 2
            + [pltpu.VMEM_SHARED((_N, 128), jnp.float32)]
            + [pltpu.SemaphoreType.DMA] * 5
        ),
    )


def _head_body(pre_ref, vidx_ref, w1_ref, b1_ref, w2_ref, b2_ref, out_ref,
               seg_ref, cnt_ref):
    i = pl.program_id(0)

    @pl.when(i == 0)
    def _():
        seg_ref[...] = jnp.zeros_like(seg_ref)
        cnt_ref[...] = jnp.zeros_like(cnt_ref)

    ids = vidx_ref[0]
    iot = lax.broadcasted_iota(jnp.int32, (_B, 1250), 0)
    maskf = (ids == iot).astype(jnp.float32)
    h0 = jnp.maximum(pre_ref[0, 0], 0.0)
    h1 = jnp.maximum(pre_ref[1, 0], 0.0)
    seg_ref[:, :128] += jnp.dot(maskf, h0, preferred_element_type=jnp.float32)
    seg_ref[:, 128:] += jnp.dot(maskf, h1, preferred_element_type=jnp.float32)
    cnt_ref[...] += jnp.broadcast_to(
        jnp.sum(maskf, axis=1, keepdims=True), (_B, 128))

    @pl.when(i == 7)
    def _():
        mean = seg_ref[...] / cnt_ref[:, :1]
        y = lax.dot_general(mean, w1_ref[...], (((1,), (1,)), ((), ())),
                            preferred_element_type=jnp.float32) + b1_ref[...]
        y = jnp.maximum(y, 0.0)
        z = lax.dot_general(y, w2_ref[...], (((1,), (1,)), ((), ())),
                            preferred_element_type=jnp.float32) + b2_ref[...]
        out_ref[...] = 1.0 / (1.0 + jnp.exp(-z))


def _head(pre, vidx, w1, b1, w2, b2):
    return pl.pallas_call(
        _head_body,
        grid=(8,),
        in_specs=[
            pl.BlockSpec((2, 1, 1250, 128), lambda i: (0, i, 0, 0)),
            pl.BlockSpec((1, 1, 1250), lambda i: (i, 0, 0)),
            pl.BlockSpec((1024, 256), lambda i: (0, 0)),
            pl.BlockSpec((1, 1024), lambda i: (0, 0)),
            pl.BlockSpec((128, 1024), lambda i: (0, 0)),
            pl.BlockSpec((1, 128), lambda i: (0, 0)),
        ],
        out_specs=pl.BlockSpec((_B, 128), lambda i: (0, 0)),
        out_shape=jax.ShapeDtypeStruct((_B, 128), jnp.float32),
        scratch_shapes=[
            pltpu.VMEM((_B, 256), jnp.float32),
            pltpu.VMEM((_B, 128), jnp.float32),
        ],
    )(pre, vidx, w1, b1, w2, b2)


def kernel(verts, edges, verts_idx, W0_0, b0_0, W1_0, b1_0, W0_1, b0_1,
           W1_1, b1_1, fc1_w, fc1_b, fc2_w, fc2_b):
    src = edges[:, 0].astype(jnp.int32)
    dst = edges[:, 1].astype(jnp.int32)
    glist = jnp.concatenate([dst, src])
    slist = jnp.concatenate([src, dst]).reshape(16, _CHUNKS // _BLK,
                                                _BLK, _K)

    w_a = jnp.stack([W0_0[:128], W0_0[128:], W1_0[:128], W1_0[128:]])
    b_a = jnp.stack([b0_0[:128], b0_0[128:], b1_0[:128], b1_0[128:]])
    b_a = b_a.reshape(4, 1, 128)
    table0 = _mm0(verts, w_a, b_a)
    pre0 = _get_sc_scatter()(table0.reshape(4 * _N, 128), glist, slist)

    w_c = jnp.stack([
        jnp.stack([W0_1[:128, :128], W0_1[:128, 128:]]),
        jnp.stack([W0_1[128:, :128], W0_1[128:, 128:]]),
        jnp.stack([W1_1[:128, :128], W1_1[:128, 128:]]),
        jnp.stack([W1_1[128:, :128], W1_1[128:, 128:]]),
    ])
    b_c = jnp.stack([b0_1[:128], b0_1[128:], b1_1[:128], b1_1[128:]])
    b_c = b_c.reshape(4, 1, 128)
    table1 = _mm1(pre0.reshape(2, _N, 128), w_c, b_c)
    pre1 = _get_sc_scatter()(table1.reshape(4 * _N, 128), glist, slist)

    fc2_wp = jnp.pad(fc2_w, ((0, 118), (0, 0)))
    fc2_bp = jnp.pad(fc2_b, (0, 118)).reshape(1, 128)
    out = _head(pre1.reshape(2, 8, 1250, 128),
                verts_idx.reshape(8, 1, 1250).astype(jnp.int32),
                fc1_w, fc1_b.reshape(1, 1024), fc2_wp, fc2_bp)
    return out[:, :10]


# speedup vs baseline: 1.4070x; 1.0021x over previous
import functools

import jax
import jax.numpy as jnp
from jax import lax
from jax.experimental import pallas as pl
from jax.experimental.pallas import tpu as pltpu
from jax.experimental.pallas import tpu_sc as plsc

_N = 10000
_E = 320000
_B = 16
_K = 80
_CHUNKS = (2 * _E) // (16 * _K)
_BLK = 25


def _mm0_body(x_ref, w_ref, b_ref, out_ref):
    out_ref[0] = lax.dot_general(
        x_ref[...], w_ref[0], (((1,), (1,)), ((), ())),
        preferred_element_type=jnp.float32) + b_ref[0]


def _mm0(x, w, b):
    return pl.pallas_call(
        _mm0_body,
        grid=(4,),
        in_specs=[
            pl.BlockSpec((_N, 128), lambda j: (0, 0)),
            pl.BlockSpec((1, 128, 128), lambda j: (j, 0, 0)),
            pl.BlockSpec((1, 1, 128), lambda j: (j, 0, 0)),
        ],
        out_specs=pl.BlockSpec((1, _N, 128), lambda j: (j, 0, 0)),
        out_shape=jax.ShapeDtypeStruct((4, _N, 128), jnp.float32),
    )(x, w, b)


def _mm1_body(pre_ref, w_ref, b_ref, out_ref):
    h0 = jnp.maximum(pre_ref[0], 0.0)
    h1 = jnp.maximum(pre_ref[1], 0.0)
    out_ref[0] = (
        lax.dot_general(h0, w_ref[0, 0], (((1,), (1,)), ((), ())),
                        preferred_element_type=jnp.float32)
        + lax.dot_general(h1, w_ref[0, 1], (((1,), (1,)), ((), ())),
                          preferred_element_type=jnp.float32)
        + b_ref[0])


def _mm1(pre, w, b):
    return pl.pallas_call(
        _mm1_body,
        grid=(4,),
        in_specs=[
            pl.BlockSpec((2, _N, 128), lambda j: (0, 0, 0)),
            pl.BlockSpec((1, 2, 128, 128), lambda j: (j, 0, 0, 0)),
            pl.BlockSpec((1, 1, 128), lambda j: (j, 0, 0)),
        ],
        out_specs=pl.BlockSpec((1, _N, 128), lambda j: (j, 0, 0)),
        out_shape=jax.ShapeDtypeStruct((4, _N, 128), jnp.float32),
    )(pre, w, b)


def _sc_scatter_body(table, glist, slist, out, gbuf0, gbuf1, sbuf0, sbuf1,
                     rows0, rows1, acc, semi, semg0, semg1, sems0, sems1):
    c = lax.axis_index("c")
    s = lax.axis_index("s")
    base_g = s * (_CHUNKS * _K)
    voff = (c + 2) * _N
    pltpu.async_copy(glist.at[pl.ds(base_g, _BLK * _K)], gbuf0, semi)
    pltpu.async_copy(slist.at[s, 0], sbuf0, semi)
    r0 = s * 624

    @pl.when(s < 15)
    def _():
        pltpu.sync_copy(table.at[pl.ds(c * _N + r0, 624)],
                        acc.at[pl.ds(r0, 624)])

    @pl.when(s == 15)
    def _():
        pltpu.sync_copy(table.at[pl.ds(c * _N + 9360, 640)],
                        acc.at[pl.ds(9360, 640)])

    plsc.subcore_barrier()

    rows = (rows0, rows1)
    semg = (semg0, semg1)
    sems = (sems0, sems1)

    def _load_block(bidx, gb, sb):
        pltpu.async_copy(glist.at[pl.ds(base_g + bidx * (_BLK * _K),
                                        _BLK * _K)], gb, semi)
        pltpu.async_copy(slist.at[s, bidx], sb, semi)

    def _wait_block(bidx, gb, sb):
        pltpu.make_async_copy(glist.at[pl.ds(base_g + bidx * (_BLK * _K),
                                             _BLK * _K)], gb, semi).wait()
        pltpu.make_async_copy(slist.at[s, bidx], sb, semi).wait()
        for l in range(_BLK * _K // 16):
            gb[pl.ds(16 * l, 16)] = gb[pl.ds(16 * l, 16)] + voff

    def _block(u, b, gb, sb, gbn, sbn):
        for k in range(_BLK):
            r = (k + b * (_BLK % 2)) % 2
            rn = 1 - r
            if k > 0:
                pltpu.make_async_copy(rows[rn], acc.at[sb.at[k - 1]],
                                      sems[rn]).wait()
            if k < _BLK - 1:
                pltpu.async_copy(table.at[gb.at[pl.ds((k + 1) * _K, _K)]],
                                 rows[rn], semg[rn])
            else:
                @pl.when((2 * u + b) < (_CHUNKS // _BLK) - 1)
                def _():
                    _wait_block(2 * u + b + 1, gbn, sbn)
                    pltpu.async_copy(table.at[gbn.at[pl.ds(0, _K)]],
                                     rows[rn], semg[rn])
            pltpu.make_async_copy(table.at[gb.at[pl.ds(k * _K, _K)]],
                                  rows[r], semg[r]).wait()
            pltpu.async_copy(rows[r], acc.at[sb.at[k]], sems[r], add=True)
        rl = (_BLK - 1 + b * (_BLK % 2)) % 2
        pltpu.make_async_copy(rows[rl], acc.at[sb.at[_BLK - 1]],
                              sems[rl]).wait()

    nblk2 = _CHUNKS // (2 * _BLK)

    _wait_block(0, gbuf0, sbuf0)
    pltpu.async_copy(table.at[gbuf0.at[pl.ds(0, _K)]], rows0, semg0)

    def step(u, carry):
        _load_block(2 * u + 1, gbuf1, sbuf1)
        _block(u, 0, gbuf0, sbuf0, gbuf1, sbuf1)

        @pl.when(u < nblk2 - 1)
        def _():
            _load_block(2 * u + 2, gbuf0, sbuf0)

        _block(u, 1, gbuf1, sbuf1, gbuf0, sbuf0)
        return carry

    lax.fori_loop(0, nblk2, step, 0)
    plsc.subcore_barrier()

    @pl.when(s < 15)
    def _():
        pltpu.sync_copy(acc.at[pl.ds(r0, 624)],
                        out.at[pl.ds(c * _N + r0, 624)])

    @pl.when(s == 15)
    def _():
        pltpu.sync_copy(acc.at[pl.ds(9360, 640)],
                        out.at[pl.ds(c * _N + 9360, 640)])


@functools.cache
def _get_sc_scatter():
    return pl.kernel(
        _sc_scatter_body,
        jax.ShapeDtypeStruct((2 * _N, 128), jnp.float32),
        mesh=plsc.VectorSubcoreMesh(core_axis_name="c", subcore_axis_name="s"),
        scratch_types=(
            [pltpu.VMEM((_BLK * _K,), jnp.int32)] * 2
            + [pltpu.VMEM((_BLK, _K), jnp.int32)] * 2
            + [pltpu.VMEM((_K, 128), jnp.float32)] * 2
            + [pltpu.VMEM_SHARED((_N, 128), jnp.float32)]
            + [pltpu.SemaphoreType.DMA] * 5
        ),
    )


def _head_body(pre_ref, vidx_ref, w1_ref, b1_ref, w2_ref, b2_ref, out_ref,
               seg_ref, cnt_ref):
    i = pl.program_id(0)

    @pl.when(i == 0)
    def _():
        seg_ref[...] = jnp.zeros_like(seg_ref)
        cnt_ref[...] = jnp.zeros_like(cnt_ref)

    ids = vidx_ref[0]
    iot = lax.broadcasted_iota(jnp.int32, (_B, 1250), 0)
    maskf = (ids == iot).astype(jnp.float32)
    h0 = jnp.maximum(pre_ref[0, 0], 0.0)
    h1 = jnp.maximum(pre_ref[1, 0], 0.0)
    seg_ref[:, :128] += jnp.dot(maskf, h0, preferred_element_type=jnp.float32)
    seg_ref[:, 128:] += jnp.dot(maskf, h1, preferred_element_type=jnp.float32)
    cnt_ref[...] += jnp.broadcast_to(
        jnp.sum(maskf, axis=1, keepdims=True), (_B, 128))

    @pl.when(i == 7)
    def _():
        mean = seg_ref[...] / cnt_ref[:, :1]
        y = lax.dot_general(mean, w1_ref[...], (((1,), (1,)), ((), ())),
                            preferred_element_type=jnp.float32) + b1_ref[...]
        y = jnp.maximum(y, 0.0)
        z = lax.dot_general(y, w2_ref[...], (((1,), (1,)), ((), ())),
                            preferred_element_type=jnp.float32) + b2_ref[...]
        out_ref[...] = 1.0 / (1.0 + jnp.exp(-z))


def _head(pre, vidx, w1, b1, w2, b2):
    return pl.pallas_call(
        _head_body,
        grid=(8,),
        in_specs=[
            pl.BlockSpec((2, 1, 1250, 128), lambda i: (0, i, 0, 0)),
            pl.BlockSpec((1, 1, 1250), lambda i: (i, 0, 0)),
            pl.BlockSpec((1024, 256), lambda i: (0, 0)),
            pl.BlockSpec((1, 1024), lambda i: (0, 0)),
            pl.BlockSpec((128, 1024), lambda i: (0, 0)),
            pl.BlockSpec((1, 128), lambda i: (0, 0)),
        ],
        out_specs=pl.BlockSpec((_B, 128), lambda i: (0, 0)),
        out_shape=jax.ShapeDtypeStruct((_B, 128), jnp.float32),
        scratch_shapes=[
            pltpu.VMEM((_B, 256), jnp.float32),
            pltpu.VMEM((_B, 128), jnp.float32),
        ],
    )(pre, vidx, w1, b1, w2, b2)


def kernel(verts, edges, verts_idx, W0_0, b0_0, W1_0, b1_0, W0_1, b0_1,
           W1_1, b1_1, fc1_w, fc1_b, fc2_w, fc2_b):
    src = edges[:, 0].astype(jnp.int32)
    dst = edges[:, 1].astype(jnp.int32)
    glist = jnp.concatenate([dst, src])
    slist = jnp.concatenate([src, dst]).reshape(16, _CHUNKS // _BLK,
                                                _BLK, _K)

    w_a = jnp.stack([W0_0[:128], W0_0[128:], W1_0[:128], W1_0[128:]])
    b_a = jnp.stack([b0_0[:128], b0_0[128:], b1_0[:128], b1_0[128:]])
    b_a = b_a.reshape(4, 1, 128)
    table0 = _mm0(verts, w_a, b_a)
    pre0 = _get_sc_scatter()(table0.reshape(4 * _N, 128), glist, slist)

    w_c = jnp.stack([
        jnp.stack([W0_1[:128, :128], W0_1[:128, 128:]]),
        jnp.stack([W0_1[128:, :128], W0_1[128:, 128:]]),
        jnp.stack([W1_1[:128, :128], W1_1[:128, 128:]]),
        jnp.stack([W1_1[128:, :128], W1_1[128:, 128:]]),
    ])
    b_c = jnp.stack([b0_1[:128], b0_1[128:], b1_1[:128], b1_1[128:]])
    b_c = b_c.reshape(4, 1, 128)
    table1 = _mm1(pre0.reshape(2, _N, 128), w_c, b_c)
    pre1 = _get_sc_scatter()(table1.reshape(4 * _N, 128), glist, slist)

    fc2_wp = jnp.pad(fc2_w, ((0, 118), (0, 0)))
    fc2_bp = jnp.pad(fc2_b, (0, 118)).reshape(1, 128)
    out = _head(pre1.reshape(2, 8, 1250, 128),
                verts_idx.reshape(8, 1, 1250).astype(jnp.int32),
                fc1_w, fc1_b.reshape(1, 1024), fc2_wp, fc2_bp)
    return out[:, :10]
